# Initial kernel scaffold; baseline (speedup 1.0000x reference)
import functools

import jax
import jax.numpy as jnp
from jax.experimental import pallas as pl
from jax.experimental.pallas import tpu as pltpu

E = 8
K = 2


def _router_body(x_ref, gw_ref, comb_ref):
    x = x_ref[...]
    logits = jax.lax.dot_general(
        x, gw_ref[...], (((1,), (1,)), ((), ())),
        preferred_element_type=jnp.float32)
    s = jax.nn.softmax(logits, axis=-1)
    lanes = jax.lax.broadcasted_iota(jnp.int32, s.shape, 1)
    m1 = jnp.max(s, axis=-1, keepdims=True)
    i1 = jnp.argmax(s, axis=-1)[:, None]
    s2 = jnp.where(lanes == i1, -jnp.inf, s)
    m2 = jnp.max(s2, axis=-1, keepdims=True)
    i2 = jnp.argmax(s2, axis=-1)[:, None]
    denom = m1 + m2
    comb = jnp.where(lanes == i1, m1 / denom, 0.0)
    comb = jnp.where(lanes == i2, m2 / denom, comb)
    comb_ref[...] = comb


def _expert_body(x_ref, comb_ref, gw_ref, uw_ref, dw_ref, out_ref):
    e = pl.program_id(1)

    @pl.when(e == 0)
    def _():
        out_ref[...] = jnp.zeros_like(out_ref)

    x = x_ref[...]
    g = jax.lax.dot_general(x, gw_ref[0], (((1,), (0,)), ((), ())),
                            preferred_element_type=jnp.float32)
    u = jax.lax.dot_general(x, uw_ref[0], (((1,), (0,)), ((), ())),
                            preferred_element_type=jnp.float32)
    h = g * jax.nn.sigmoid(g) * u
    d = jax.lax.dot_general(h, dw_ref[0], (((1,), (0,)), ((), ())),
                            preferred_element_type=jnp.float32)
    out_ref[...] += comb_ref[...] * d


def kernel(hidden_states, gate_weight, gate_proj_w, up_proj_w, down_proj_w):
    b, s, h = hidden_states.shape
    x = hidden_states.reshape(-1, h)
    t = x.shape[0]
    f = gate_proj_w.shape[2]
    bt = 256
    nb = t // bt

    comb = pl.pallas_call(
        _router_body,
        grid=(nb,),
        in_specs=[
            pl.BlockSpec((bt, h), lambda i: (i, 0)),
            pl.BlockSpec((E, h), lambda i: (0, 0)),
        ],
        out_specs=pl.BlockSpec((bt, E), lambda i: (i, 0)),
        out_shape=jax.ShapeDtypeStruct((t, E), jnp.float32),
    )(x, gate_weight)

    out = pl.pallas_call(
        _expert_body,
        grid=(nb, E),
        in_specs=[
            pl.BlockSpec((bt, h), lambda i, e: (i, 0)),
            pl.BlockSpec((bt, 1), lambda i, e: (i, e)),
            pl.BlockSpec((1, h, f), lambda i, e: (e, 0, 0)),
            pl.BlockSpec((1, h, f), lambda i, e: (e, 0, 0)),
            pl.BlockSpec((1, f, h), lambda i, e: (e, 0, 0)),
        ],
        out_specs=pl.BlockSpec((bt, h), lambda i, e: (i, 0)),
        out_shape=jax.ShapeDtypeStruct((t, h), jnp.float32),
        compiler_params=pltpu.CompilerParams(
            dimension_semantics=("parallel", "arbitrary")),
    )(x, comb, gate_proj_w, up_proj_w, down_proj_w)

    return out.reshape(b, s, h)


# speedup vs baseline: 1.2858x; 1.2858x over previous
import functools

import jax
import jax.numpy as jnp
from jax.experimental import pallas as pl
from jax.experimental.pallas import tpu as pltpu

E = 8
K = 2


def _router_body(x_ref, gw_ref, comb_ref):
    x = x_ref[...]
    logits = jax.lax.dot_general(
        x, gw_ref[...], (((1,), (1,)), ((), ())),
        preferred_element_type=jnp.float32)
    s = jax.nn.softmax(logits, axis=-1)
    lanes = jax.lax.broadcasted_iota(jnp.int32, s.shape, 1)
    m1 = jnp.max(s, axis=-1, keepdims=True)
    i1 = jnp.argmax(s, axis=-1)[:, None]
    s2 = jnp.where(lanes == i1, -jnp.inf, s)
    m2 = jnp.max(s2, axis=-1, keepdims=True)
    i2 = jnp.argmax(s2, axis=-1)[:, None]
    denom = m1 + m2
    comb = jnp.where(lanes == i1, m1 / denom, 0.0)
    comb = jnp.where(lanes == i2, m2 / denom, comb)
    comb_ref[...] = comb


def _expert_body(x_ref, comb_ref, gw_ref, uw_ref, dw_ref, out_ref):
    e = pl.program_id(1)

    @pl.when(e == 0)
    def _():
        out_ref[...] = jnp.zeros_like(out_ref)

    x = x_ref[...]
    g = jax.lax.dot_general(x, gw_ref[0], (((1,), (0,)), ((), ())),
                            preferred_element_type=jnp.float32)
    u = jax.lax.dot_general(x, uw_ref[0], (((1,), (0,)), ((), ())),
                            preferred_element_type=jnp.float32)
    h = g * jax.nn.sigmoid(g) * u
    d = jax.lax.dot_general(h, dw_ref[0], (((1,), (0,)), ((), ())),
                            preferred_element_type=jnp.float32)
    comb = comb_ref[...]
    lanes = jax.lax.broadcasted_iota(jnp.int32, comb.shape, 1)
    col = jnp.sum(jnp.where(lanes == e, comb, 0.0), axis=1, keepdims=True)
    out_ref[...] += col * d


def kernel(hidden_states, gate_weight, gate_proj_w, up_proj_w, down_proj_w):
    b, s, h = hidden_states.shape
    x = hidden_states.reshape(-1, h)
    t = x.shape[0]
    f = gate_proj_w.shape[2]
    bt = 256
    nb = t // bt

    comb = pl.pallas_call(
        _router_body,
        grid=(nb,),
        in_specs=[
            pl.BlockSpec((bt, h), lambda i: (i, 0)),
            pl.BlockSpec((E, h), lambda i: (0, 0)),
        ],
        out_specs=pl.BlockSpec((bt, E), lambda i: (i, 0)),
        out_shape=jax.ShapeDtypeStruct((t, E), jnp.float32),
    )(x, gate_weight)

    out = pl.pallas_call(
        _expert_body,
        grid=(nb, E),
        in_specs=[
            pl.BlockSpec((bt, h), lambda i, e: (i, 0)),
            pl.BlockSpec((bt, E), lambda i, e: (i, 0)),
            pl.BlockSpec((1, h, f), lambda i, e: (e, 0, 0)),
            pl.BlockSpec((1, h, f), lambda i, e: (e, 0, 0)),
            pl.BlockSpec((1, f, h), lambda i, e: (e, 0, 0)),
        ],
        out_specs=pl.BlockSpec((bt, h), lambda i, e: (i, 0)),
        out_shape=jax.ShapeDtypeStruct((t, h), jnp.float32),
        compiler_params=pltpu.CompilerParams(
            dimension_semantics=("parallel", "arbitrary")),
    )(x, comb, gate_proj_w, up_proj_w, down_proj_w)

    return out.reshape(b, s, h)
